# baseline (speedup 1.0000x reference)
import functools

import jax
import jax.numpy as jnp
from jax.experimental import pallas as pl
from jax.experimental.pallas import tpu as pltpu

_LO = 64


def _count_kernel(src_ref, dst_ref, out_ref, acc_a, acc_b, smem_s, smem_d,
                  sem, *, nt, tb, n_edges, padded):
    i32 = jnp.int32

    @pl.when(pl.program_id(1) == 0)
    def _():
        acc_a[...] = jnp.zeros_like(acc_a)
        acc_b[...] = jnp.zeros_like(acc_b)

    pltpu.make_async_copy(src_ref, smem_s, sem.at[0]).start()
    pltpu.make_async_copy(dst_ref, smem_d, sem.at[1]).start()
    pltpu.make_async_copy(src_ref, smem_s, sem.at[0]).wait()
    pltpu.make_async_copy(dst_ref, smem_d, sem.at[1]).wait()

    sub8 = jax.lax.broadcasted_iota(i32, (8, 128), 0)
    lane128 = jax.lax.broadcasted_iota(i32, (8, 128), 1)
    if padded:
        step = pl.program_id(0) * nt + pl.program_id(1)

    def body(k, carry):
        base = k * 8
        infos = []
        for u in range(8):
            i = base + u
            s = smem_s[0, i]
            d = smem_d[0, i]
            rb = pl.multiple_of((s >> 3) << 3, 8)
            cb = pl.multiple_of((d >> 7) << 7, 128)
            key = ((s >> 3) << 4) + (d >> 7)
            m = (sub8 == (s & 7)) & (lane128 == (d & 127))
            if padded:
                m = m & (step * tb + i < n_edges)
            infos.append((rb, cb, key, m.astype(jnp.float32)))
        for ref, group in ((acc_a, infos[0::2]), (acc_b, infos[1::2])):
            loads = [ref[pl.ds(g[0], 8), pl.ds(g[1], 128)] for g in group]
            upds = []
            for j in range(4):
                mj = group[j][3]
                for i2 in range(j):
                    pred = (group[i2][2] == group[j][2]).astype(jnp.float32)
                    mj = mj + pred * group[i2][3]
                upds.append(loads[j] + mj)
            for j in range(4):
                ref[pl.ds(group[j][0], 8), pl.ds(group[j][1], 128)] = upds[j]
        return carry

    jax.lax.fori_loop(0, tb // 8, body, 0)

    @pl.when(pl.program_id(1) == nt - 1)
    def _():
        acc_a[...] = acc_a[...] + acc_b[...]
        cp = pltpu.make_async_copy(acc_a, out_ref.at[pl.program_id(0)],
                                   sem.at[0])
        cp.start()
        cp.wait()


def _node_kernel(at_ref, xt_ref, w1t_ref, b1_ref, w2t_ref, b2_ref,
                 wheadt_ref, y_ref):
    f32 = jnp.float32
    a_t = at_ref[...]
    xa = jnp.dot(xt_ref[...], a_t, preferred_element_type=f32)
    h1 = jnp.maximum(
        jnp.dot(w1t_ref[...], xa, preferred_element_type=f32) + b1_ref[...], 0.0)
    h1w = jnp.dot(w2t_ref[...], h1, preferred_element_type=f32)
    h2 = jnp.maximum(
        jnp.dot(h1w, a_t, preferred_element_type=f32) + b2_ref[...], 0.0)
    y_ref[...] = jnp.dot(wheadt_ref[...], h2, preferred_element_type=f32)


def _edge_kernel(wt_ref, src_ref, dst_ref, attr_ref, wattr_ref, blin_ref,
                 out_ref):
    f32 = jnp.float32
    te = src_ref.shape[1]
    hi = wt_ref.shape[0] // 4

    s = src_ref[...]
    d = dst_ref[...]
    sub_lo = jax.lax.broadcasted_iota(jnp.int32, (_LO, te), 0)
    oh_s = (sub_lo == (s & (_LO - 1))).astype(f32)
    oh_d = (sub_lo == (d & (_LO - 1))).astype(f32)
    g = jnp.concatenate([oh_s, oh_d], axis=0)
    t = jnp.dot(wt_ref[...], g, preferred_element_type=f32)

    sub_hi = jax.lax.broadcasted_iota(jnp.int32, (hi, te), 0)
    hs = (sub_hi == (s >> 6)).astype(f32)
    hd = (sub_hi == (d >> 6)).astype(f32)
    a = attr_ref[...]
    o0 = (jnp.sum(t[0:hi] * hs, axis=0, keepdims=True)
          + jnp.sum(t[2 * hi:3 * hi] * hd, axis=0, keepdims=True)
          + jnp.sum(a * wattr_ref[:, 0:1], axis=0, keepdims=True))
    o1 = (jnp.sum(t[hi:2 * hi] * hs, axis=0, keepdims=True)
          + jnp.sum(t[3 * hi:4 * hi] * hd, axis=0, keepdims=True)
          + jnp.sum(a * wattr_ref[:, 1:2], axis=0, keepdims=True))
    out_ref[...] = jnp.concatenate([o0, o1], axis=0) + blin_ref[...]


@functools.partial(jax.jit, static_argnames=("edge_tile",))
def _forward(x, edge_index, edge_attr, W1, b1, W2, b2, Wlin, blin,
             edge_tile=4096):
    f32 = jnp.float32
    N, F_IN = x.shape
    E = edge_index.shape[1]
    H1 = W1.shape[1]
    H2 = W2.shape[1]
    A_DIM = edge_attr.shape[1]
    HI = -(-N // _LO)

    src, dst = edge_index[0].astype(jnp.int32), edge_index[1].astype(jnp.int32)
    e_pad = -(-E // (2 * edge_tile)) * (2 * edge_tile)
    if e_pad != E:
        src_r = jnp.zeros((1, e_pad), jnp.int32).at[0, :E].set(src)
        dst_r = jnp.zeros((1, e_pad), jnp.int32).at[0, :E].set(dst)
        attr_t = jnp.zeros((A_DIM, e_pad), f32).at[:, :E].set(edge_attr.T.astype(f32))
    else:
        src_r = src.reshape(1, E)
        dst_r = dst.reshape(1, E)
        attr_t = edge_attr.T.astype(f32)
    nt = e_pad // (2 * edge_tile)

    counts = pl.pallas_call(
        functools.partial(_count_kernel, nt=nt, tb=edge_tile, n_edges=E,
                          padded=(e_pad != E)),
        out_shape=jax.ShapeDtypeStruct((2, N, N), f32),
        grid=(2, nt),
        in_specs=[
            pl.BlockSpec((1, edge_tile), lambda c, t: (0, c * nt + t)),
            pl.BlockSpec((1, edge_tile), lambda c, t: (0, c * nt + t)),
        ],
        out_specs=pl.BlockSpec(memory_space=pl.ANY),
        scratch_shapes=[
            pltpu.VMEM((N, N), f32),
            pltpu.VMEM((N, N), f32),
            pltpu.SMEM((1, edge_tile), jnp.int32),
            pltpu.SMEM((1, edge_tile), jnp.int32),
            pltpu.SemaphoreType.DMA((2,)),
        ],
        compiler_params=pltpu.CompilerParams(
            dimension_semantics=("parallel", "arbitrary")),
    )(src_r, dst_r)
    at = counts[0] + counts[1] + jnp.eye(N, dtype=f32)
    dinv = jax.lax.rsqrt(jnp.sum(at, axis=0))
    a_hat_t = at * dinv[:, None] * dinv[None, :]

    w_src = Wlin[:H2]
    w_attr = Wlin[H2:H2 + A_DIM]
    w_dst = Wlin[H2 + A_DIM:]
    w_head_t = jnp.concatenate([w_src, w_dst], axis=1).T.astype(f32)

    y = pl.pallas_call(
        _node_kernel,
        out_shape=jax.ShapeDtypeStruct((4, N), f32),
        grid=(1,),
        in_specs=[
            pl.BlockSpec((N, N), lambda i: (0, 0)),
            pl.BlockSpec((F_IN, N), lambda i: (0, 0)),
            pl.BlockSpec((H1, F_IN), lambda i: (0, 0)),
            pl.BlockSpec((H1, 1), lambda i: (0, 0)),
            pl.BlockSpec((H2, H1), lambda i: (0, 0)),
            pl.BlockSpec((H2, 1), lambda i: (0, 0)),
            pl.BlockSpec((4, H2), lambda i: (0, 0)),
        ],
        out_specs=pl.BlockSpec((4, N), lambda i: (0, 0)),
        compiler_params=pltpu.CompilerParams(dimension_semantics=("arbitrary",)),
    )(a_hat_t, x.T.astype(f32), W1.T.astype(f32), b1.reshape(-1, 1).astype(f32),
      W2.T.astype(f32), b2.reshape(-1, 1).astype(f32), w_head_t)

    y_pad = jnp.zeros((4, HI * _LO), f32).at[:, :N].set(y) if HI * _LO != N else y
    wt = (jnp.zeros((4 * HI, 2 * _LO), f32)
          .at[:2 * HI, :_LO].set(y_pad[0:2].reshape(2 * HI, _LO))
          .at[2 * HI:, _LO:].set(y_pad[2:4].reshape(2 * HI, _LO)))

    out_t = pl.pallas_call(
        _edge_kernel,
        out_shape=jax.ShapeDtypeStruct((2, e_pad), f32),
        grid=(2, nt),
        in_specs=[
            pl.BlockSpec((4 * HI, 2 * _LO), lambda c, t: (0, 0)),
            pl.BlockSpec((1, edge_tile), lambda c, t: (0, c * nt + t)),
            pl.BlockSpec((1, edge_tile), lambda c, t: (0, c * nt + t)),
            pl.BlockSpec((A_DIM, edge_tile), lambda c, t: (0, c * nt + t)),
            pl.BlockSpec((A_DIM, 2), lambda c, t: (0, 0)),
            pl.BlockSpec((2, 1), lambda c, t: (0, 0)),
        ],
        out_specs=pl.BlockSpec((2, edge_tile), lambda c, t: (0, c * nt + t)),
        compiler_params=pltpu.CompilerParams(
            dimension_semantics=("parallel", "arbitrary")),
    )(wt, src_r, dst_r, attr_t, w_attr.astype(f32), blin.reshape(2, 1).astype(f32))

    return out_t[:, :E].T


def kernel(x, edge_index, edge_attr, W1, b1, W2, b2, Wlin, blin):
    return _forward(x, edge_index, edge_attr, W1, b1, W2, b2, Wlin, blin,
                    edge_tile=4096)


# speedup vs baseline: 2.8654x; 2.8654x over previous
import functools

import jax
import jax.numpy as jnp
from jax.experimental import pallas as pl
from jax.experimental.pallas import tpu as pltpu

_LO = 64


def _node_kernel(at_ref, xt_ref, w1t_ref, b1_ref, w2t_ref, b2_ref,
                 wheadt_ref, y_ref):
    f32 = jnp.float32
    a_t = at_ref[...]
    xa = jnp.dot(xt_ref[...], a_t, preferred_element_type=f32)
    h1 = jnp.maximum(
        jnp.dot(w1t_ref[...], xa, preferred_element_type=f32) + b1_ref[...], 0.0)
    h1w = jnp.dot(w2t_ref[...], h1, preferred_element_type=f32)
    h2 = jnp.maximum(
        jnp.dot(h1w, a_t, preferred_element_type=f32) + b2_ref[...], 0.0)
    y_ref[...] = jnp.dot(wheadt_ref[...], h2, preferred_element_type=f32)


def _edge_kernel(wt_ref, src_ref, dst_ref, attr_ref, wattr_ref, blin_ref,
                 out_ref):
    f32 = jnp.float32
    te = src_ref.shape[1]
    hi = wt_ref.shape[0] // 4

    s = src_ref[...]
    d = dst_ref[...]
    sub_lo = jax.lax.broadcasted_iota(jnp.int32, (_LO, te), 0)
    oh_s = (sub_lo == (s & (_LO - 1))).astype(f32)
    oh_d = (sub_lo == (d & (_LO - 1))).astype(f32)
    g = jnp.concatenate([oh_s, oh_d], axis=0)
    t = jnp.dot(wt_ref[...], g, preferred_element_type=f32)

    sub_hi = jax.lax.broadcasted_iota(jnp.int32, (hi, te), 0)
    hs = sub_hi == (s >> 6)
    hd = sub_hi == (d >> 6)
    zero = jnp.zeros((), f32)
    a = attr_ref[...]
    o0 = (jnp.sum(jnp.where(hs, t[0:hi], zero), axis=0, keepdims=True)
          + jnp.sum(jnp.where(hd, t[2 * hi:3 * hi], zero), axis=0, keepdims=True)
          + jnp.sum(a * wattr_ref[:, 0:1], axis=0, keepdims=True))
    o1 = (jnp.sum(jnp.where(hs, t[hi:2 * hi], zero), axis=0, keepdims=True)
          + jnp.sum(jnp.where(hd, t[3 * hi:4 * hi], zero), axis=0, keepdims=True)
          + jnp.sum(a * wattr_ref[:, 1:2], axis=0, keepdims=True))
    out_ref[...] = jnp.concatenate([o0, o1], axis=0) + blin_ref[...]


@functools.partial(jax.jit, static_argnames=("edge_tile",))
def _forward(x, edge_index, edge_attr, W1, b1, W2, b2, Wlin, blin,
             edge_tile=4096):
    f32 = jnp.float32
    N, F_IN = x.shape
    E = edge_index.shape[1]
    H1 = W1.shape[1]
    H2 = W2.shape[1]
    A_DIM = edge_attr.shape[1]
    HI = -(-N // _LO)

    src, dst = edge_index[0].astype(jnp.int32), edge_index[1].astype(jnp.int32)
    half = E // 2
    at0 = jnp.zeros((N, N), f32).at[src[:half], dst[:half]].add(1.0)
    at1 = jnp.zeros((N + 8, N), f32).at[src[half:], dst[half:]].add(1.0)
    at = at0 + at1[:N] + jnp.eye(N, dtype=f32)
    dinv = jax.lax.rsqrt(jnp.sum(at, axis=0))
    a_hat_t = at * dinv[:, None] * dinv[None, :]

    w_src = Wlin[:H2]
    w_attr = Wlin[H2:H2 + A_DIM]
    w_dst = Wlin[H2 + A_DIM:]
    w_head_t = jnp.concatenate([w_src, w_dst], axis=1).T.astype(f32)

    y = pl.pallas_call(
        _node_kernel,
        out_shape=jax.ShapeDtypeStruct((4, N), f32),
        grid=(1,),
        in_specs=[
            pl.BlockSpec((N, N), lambda i: (0, 0)),
            pl.BlockSpec((F_IN, N), lambda i: (0, 0)),
            pl.BlockSpec((H1, F_IN), lambda i: (0, 0)),
            pl.BlockSpec((H1, 1), lambda i: (0, 0)),
            pl.BlockSpec((H2, H1), lambda i: (0, 0)),
            pl.BlockSpec((H2, 1), lambda i: (0, 0)),
            pl.BlockSpec((4, H2), lambda i: (0, 0)),
        ],
        out_specs=pl.BlockSpec((4, N), lambda i: (0, 0)),
        compiler_params=pltpu.CompilerParams(dimension_semantics=("arbitrary",)),
    )(a_hat_t, x.T.astype(f32), W1.T.astype(f32), b1.reshape(-1, 1).astype(f32),
      W2.T.astype(f32), b2.reshape(-1, 1).astype(f32), w_head_t)

    y_pad = jnp.zeros((4, HI * _LO), f32).at[:, :N].set(y) if HI * _LO != N else y
    wt = (jnp.zeros((4 * HI, 2 * _LO), f32)
          .at[:2 * HI, :_LO].set(y_pad[0:2].reshape(2 * HI, _LO))
          .at[2 * HI:, _LO:].set(y_pad[2:4].reshape(2 * HI, _LO)))

    e_pad = -(-E // (2 * edge_tile)) * (2 * edge_tile)
    if e_pad != E:
        src_r = jnp.zeros((1, e_pad), jnp.int32).at[0, :E].set(src)
        dst_r = jnp.zeros((1, e_pad), jnp.int32).at[0, :E].set(dst)
        attr_t = jnp.zeros((A_DIM, e_pad), f32).at[:, :E].set(edge_attr.T.astype(f32))
    else:
        src_r = src.reshape(1, E)
        dst_r = dst.reshape(1, E)
        attr_t = edge_attr.T.astype(f32)

    nt = e_pad // (2 * edge_tile)
    out_t = pl.pallas_call(
        _edge_kernel,
        out_shape=jax.ShapeDtypeStruct((2, e_pad), f32),
        grid=(2, nt),
        in_specs=[
            pl.BlockSpec((4 * HI, 2 * _LO), lambda c, t: (0, 0)),
            pl.BlockSpec((1, edge_tile), lambda c, t: (0, c * nt + t)),
            pl.BlockSpec((1, edge_tile), lambda c, t: (0, c * nt + t)),
            pl.BlockSpec((A_DIM, edge_tile), lambda c, t: (0, c * nt + t)),
            pl.BlockSpec((A_DIM, 2), lambda c, t: (0, 0)),
            pl.BlockSpec((2, 1), lambda c, t: (0, 0)),
        ],
        out_specs=pl.BlockSpec((2, edge_tile), lambda c, t: (0, c * nt + t)),
        compiler_params=pltpu.CompilerParams(
            dimension_semantics=("parallel", "arbitrary")),
    )(wt, src_r, dst_r, attr_t, w_attr.astype(f32), blin.reshape(2, 1).astype(f32))

    return out_t[:, :E].T


def kernel(x, edge_index, edge_attr, W1, b1, W2, b2, Wlin, blin):
    return _forward(x, edge_index, edge_attr, W1, b1, W2, b2, Wlin, blin,
                    edge_tile=8192)
